# baseline (speedup 1.0000x reference)
import functools

import jax
import jax.numpy as jnp
from jax import lax
from jax.experimental import pallas as pl
from jax.experimental.pallas import tpu as pltpu
from jax.experimental.pallas import tpu_sc as plsc

_B, _S, _D = 4096, 200, 300
_V = 100002
_VPAD = 100008
_DPAD = 384
_TB_ROWS = _VPAD * 3
_NBT = _B // 128
_GROUPS = _S * _NBT
_ND2 = _D - 256


@functools.cache
def _make_gather():
    info = plsc.get_sparse_core_info()
    nc, ns = info.num_cores, info.num_subcores
    nw = nc * ns
    per_w = _GROUPS // nw
    n_tok = per_w * 128
    mesh = plsc.VectorSubcoreMesh(core_axis_name="c", subcore_axis_name="s")

    @functools.partial(
        pl.kernel,
        mesh=mesh,
        compiler_params=pltpu.CompilerParams(needs_layout_passes=False),
        out_type=jax.ShapeDtypeStruct((_D, _S, _B), jnp.float32),
        scratch_types=[
            pltpu.VMEM((n_tok,), jnp.int32),
            pltpu.VMEM((128,), jnp.int32),
            pltpu.VMEM((128,), jnp.int32),
            pltpu.VMEM((128, 129), jnp.float32),
            pltpu.VMEM((128, 129), jnp.float32),
            pltpu.VMEM((128, 128), jnp.float32),
            pltpu.VMEM((128, 128), jnp.float32),
            pltpu.SemaphoreType.DMA,
            pltpu.SemaphoreType.DMA,
            pltpu.SemaphoreType.DMA,
            pltpu.SemaphoreType.DMA,
        ],
    )
    def gather(ids_hbm, tb2_hbm, out_hbm, ids_my, blk_a, blk_b,
               gbuf_a, gbuf_b, tbuf_a, tbuf_b,
               in_a, in_b, out_a, out_b):
        wid = lax.axis_index("s") * nc + lax.axis_index("c")
        lane = lax.iota(jnp.int32, 16)

        pltpu.sync_copy(ids_hbm.at[pl.ds(wid * n_tok, n_tok)], ids_my)

        def compute_blk(i, dst, kofs):
            for jg in range(8):
                v16 = ids_my[pl.ds(i * 128 + jg * 16, 16)]
                dst[pl.ds(jg * 16, 16)] = (v16 >> 3) * 24 + (v16 & 7) + kofs

        def bump_blk(src, dst):
            for jg in range(8):
                dst[pl.ds(jg * 16, 16)] = src[pl.ds(jg * 16, 16)] + 8

        def fire_in(blk, gbuf, sem):
            pltpu.async_copy(tb2_hbm.at[blk], gbuf.at[:, pl.ds(0, 128)], sem)

        def wait_in(blk, gbuf, sem):
            pltpu.make_async_copy(
                tb2_hbm.at[blk], gbuf.at[:, pl.ds(0, 128)], sem
            ).wait()

        def out_slice(i, k, nd):
            g = wid * per_w + i
            s = g >> 5
            bt = g & 31
            return out_hbm.at[pl.ds(k * 128, nd), s, pl.ds(bt * 128, 128)]

        def fire_out(tbuf, i, k, nd, sem):
            pltpu.async_copy(tbuf.at[pl.ds(0, nd)], out_slice(i, k, nd), sem)

        def wait_out(tbuf, i, k, nd, sem):
            pltpu.make_async_copy(
                tbuf.at[pl.ds(0, nd)], out_slice(i, k, nd), sem
            ).wait()

        def transpose(gbuf, tbuf, nd):
            ngrp = nd // 16
            @plsc.parallel_loop(0, 128, 1, unroll=4)
            def body(j):
                jvec = jnp.full((16,), 0, jnp.int32) + j
                vals = [gbuf[j, pl.ds(jg * 16, 16)] for jg in range(ngrp)]
                for jg in range(ngrp):
                    plsc.store_scatter(
                        tbuf, [lane + jg * 16, jvec], vals[jg]
                    )

        compute_blk(0, blk_a, 0)
        fire_in(blk_a, gbuf_a, in_a)

        def loop01(i, carry):
            wait_in(blk_a, gbuf_a, in_a)
            bump_blk(blk_a, blk_b)
            fire_in(blk_b, gbuf_b, in_b)

            @pl.when(i != 0)
            def _():
                wait_out(tbuf_a, i - 1, 0, 128, out_a)

            transpose(gbuf_a, tbuf_a, 128)
            fire_out(tbuf_a, i, 0, 128, out_a)

            @pl.when(i + 1 < per_w)
            def _():
                compute_blk(i + 1, blk_a, 0)
                fire_in(blk_a, gbuf_a, in_a)

            wait_in(blk_b, gbuf_b, in_b)

            @pl.when(i != 0)
            def _():
                wait_out(tbuf_b, i - 1, 1, 128, out_b)

            transpose(gbuf_b, tbuf_b, 128)
            fire_out(tbuf_b, i, 1, 128, out_b)
            return carry

        lax.fori_loop(0, per_w, loop01, 0)

        wait_out(tbuf_a, per_w - 1, 0, 128, out_a)
        wait_out(tbuf_b, per_w - 1, 1, 128, out_b)

        compute_blk(0, blk_a, 16)
        fire_in(blk_a, gbuf_a, in_a)

        def loop2(j, carry):
            i0 = 2 * j
            i1 = 2 * j + 1
            wait_in(blk_a, gbuf_a, in_a)
            compute_blk(i1, blk_b, 16)
            fire_in(blk_b, gbuf_b, in_b)

            @pl.when(j != 0)
            def _():
                wait_out(tbuf_a, i0 - 2, 2, _ND2, out_a)

            transpose(gbuf_a, tbuf_a, 48)
            fire_out(tbuf_a, i0, 2, _ND2, out_a)

            @pl.when(i0 + 2 < per_w)
            def _():
                compute_blk(i0 + 2, blk_a, 16)
                fire_in(blk_a, gbuf_a, in_a)

            wait_in(blk_b, gbuf_b, in_b)

            @pl.when(j != 0)
            def _():
                wait_out(tbuf_b, i1 - 2, 2, _ND2, out_b)

            transpose(gbuf_b, tbuf_b, 48)
            fire_out(tbuf_b, i1, 2, _ND2, out_b)
            return carry

        lax.fori_loop(0, per_w // 2, loop2, 0)

        wait_out(tbuf_a, per_w - 2, 2, _ND2, out_a)
        wait_out(tbuf_b, per_w - 1, 2, _ND2, out_b)

    return gather


def kernel(token_ids, table):
    ids_t = token_ids.T.astype(jnp.int32).reshape(-1)
    tb2 = (
        jnp.pad(table, ((0, _VPAD - _V), (0, _DPAD - _D)))
        .reshape(_VPAD // 8, 8, 3, 128)
        .transpose(0, 2, 1, 3)
        .reshape(_TB_ROWS, 128)
    )
    out = _make_gather()(ids_t, tb2)
    return out.transpose(2, 1, 0)


# speedup vs baseline: 2.1928x; 2.1928x over previous
import functools

import jax
import jax.numpy as jnp
from jax import lax
from jax.experimental import pallas as pl
from jax.experimental.pallas import tpu as pltpu
from jax.experimental.pallas import tpu_sc as plsc

_D = 300
_D_PAD = 384
_CHUNK = 128


@functools.cache
def _make_gather(n_tokens):
    info = plsc.get_sparse_core_info()
    nc, ns = info.num_cores, info.num_subcores
    nw = nc * ns
    assert n_tokens % (nw * 2 * _CHUNK) == 0, n_tokens
    per_w = n_tokens // nw
    n_pairs = per_w // (2 * _CHUNK)
    mesh = plsc.VectorSubcoreMesh(core_axis_name="c", subcore_axis_name="s")

    @functools.partial(
        pl.kernel,
        mesh=mesh,
        out_type=jax.ShapeDtypeStruct((n_tokens, _D_PAD), jnp.float32),
        scratch_types=[
            pltpu.VMEM((per_w,), jnp.int32),
            pltpu.VMEM((_CHUNK, _D_PAD), jnp.float32),
            pltpu.VMEM((_CHUNK, _D_PAD), jnp.float32),
            pltpu.SemaphoreType.DMA,
            pltpu.SemaphoreType.DMA,
            pltpu.SemaphoreType.DMA,
            pltpu.SemaphoreType.DMA,
        ],
    )
    def gather(idx_hbm, table_hbm, out_hbm, ids_my, rows_a, rows_b,
               in_a, in_b, out_a, out_b):
        wid = lax.axis_index("s") * nc + lax.axis_index("c")
        base = wid * per_w

        pltpu.sync_copy(idx_hbm.at[pl.ds(base, per_w)], ids_my)

        def idx_slice(c):
            return ids_my.at[pl.ds(c * _CHUNK, _CHUNK)]

        def out_slice(c):
            return out_hbm.at[pl.ds(base + c * _CHUNK, _CHUNK)]

        def fire_in(c, rows, sem):
            pltpu.async_copy(table_hbm.at[idx_slice(c)], rows, sem)

        def wait_in(c, rows, sem):
            pltpu.make_async_copy(table_hbm.at[idx_slice(c)], rows, sem).wait()

        def fire_out(c, rows, sem):
            pltpu.async_copy(rows, out_slice(c), sem)

        def wait_out(c, rows, sem):
            pltpu.make_async_copy(rows, out_slice(c), sem).wait()

        fire_in(0, rows_a, in_a)

        def pair(j, carry):
            c0 = 2 * j
            c1 = 2 * j + 1

            @pl.when(j != 0)
            def _():
                wait_out(c1 - 2, rows_b, out_b)

            fire_in(c1, rows_b, in_b)
            wait_in(c0, rows_a, in_a)
            fire_out(c0, rows_a, out_a)

            @pl.when(j + 1 < n_pairs)
            def _():
                wait_out(c0, rows_a, out_a)
                fire_in(c0 + 2, rows_a, in_a)

            wait_in(c1, rows_b, in_b)
            fire_out(c1, rows_b, out_b)
            return carry

        lax.fori_loop(0, n_pairs, pair, 0)

        wait_out(per_w // _CHUNK - 2, rows_a, out_a)
        wait_out(per_w // _CHUNK - 1, rows_b, out_b)

    return gather


def kernel(token_ids, table):
    flat = token_ids.reshape(-1).astype(jnp.int32)
    table_pad = jnp.pad(table, ((0, 0), (0, _D_PAD - _D)))
    out = _make_gather(flat.shape[0])(flat, table_pad)
    return out[:, :_D].reshape(*token_ids.shape, table.shape[1])
